# baseline (speedup 1.0000x reference)
import functools

import jax
import jax.numpy as jnp
from jax import lax
from jax.experimental import pallas as pl
from jax.experimental.pallas import tpu as pltpu
from jax.experimental.pallas import tpu_sc as plsc

EMBED = 16


@functools.lru_cache(maxsize=None)
def _build(n_idx, vocab):
    info = plsc.get_sparse_core_info()
    nw = info.num_cores * info.num_subcores
    b_per_w = n_idx // nw
    n_slots = 4
    chunk = 1280
    n_chunks = b_per_w // chunk

    mesh = plsc.VectorSubcoreMesh(core_axis_name="c", subcore_axis_name="s")

    scratch = [
        pltpu.VMEM((b_per_w,), jnp.int32),
        pltpu.VMEM((n_slots, chunk, EMBED), jnp.float32),
    ] + [pltpu.SemaphoreType.DMA] * (2 * n_slots)

    @functools.partial(
        pl.kernel,
        out_type=jax.ShapeDtypeStruct((n_idx, EMBED), jnp.float32),
        mesh=mesh,
        scratch_types=scratch,
        compiler_params=pltpu.CompilerParams(use_tc_tiling_on_sc=False),
    )
    def emb_kernel(idx_hbm, table_hbm, out_hbm, idx_v, rows_v, *sems):
        g_sems = sems[:n_slots]
        o_sems = sems[n_slots:]
        wid = lax.axis_index("s") * info.num_cores + lax.axis_index("c")
        base = wid * b_per_w
        pltpu.sync_copy(idx_hbm.at[pl.ds(base, b_per_w)], idx_v)

        gather_cp = [None] * n_slots
        out_cp = [None] * n_slots

        def gather_start(g, s):
            return pltpu.async_copy(
                table_hbm.at[idx_v.at[pl.ds(g * chunk, chunk)]],
                rows_v.at[s],
                g_sems[s],
            )

        def out_start(g, s):
            return pltpu.async_copy(
                rows_v.at[s],
                out_hbm.at[pl.ds(base + g * chunk, chunk)],
                o_sems[s],
            )

        depth = min(n_slots - 1, n_chunks)
        for g in range(n_chunks):
            s = g % n_slots
            if out_cp[s] is not None:
                out_cp[s].wait()
            gather_cp[s] = gather_start(g, s)
            d = g - depth
            if d >= 0:
                sd = d % n_slots
                gather_cp[sd].wait()
                out_cp[sd] = out_start(d, sd)
        for d in range(max(0, n_chunks - depth), n_chunks):
            sd = d % n_slots
            gather_cp[sd].wait()
            out_cp[sd] = out_start(d, sd)
        for s in range(n_slots):
            if out_cp[s] is not None:
                out_cp[s].wait()

    return emb_kernel


def kernel(x, table):
    batch, _, hist = x.shape
    selected = x[:, 0, :].reshape(-1)
    fn = _build(selected.shape[0], table.shape[0])
    out = fn(selected, table)
    return out.reshape(batch, hist, EMBED)


# speedup vs baseline: 1.3037x; 1.3037x over previous
import functools

import jax
import jax.numpy as jnp
from jax import lax
from jax.experimental import pallas as pl
from jax.experimental.pallas import tpu as pltpu
from jax.experimental.pallas import tpu_sc as plsc

EMBED = 16


@functools.lru_cache(maxsize=None)
def _build(hist, batch, vocab):
    info = plsc.get_sparse_core_info()
    nw = info.num_cores * info.num_subcores
    bq = 1024
    n_bq = batch // bq
    n_tasks_total = hist * n_bq
    n_tasks = n_tasks_total // nw

    mesh = plsc.VectorSubcoreMesh(core_axis_name="c", subcore_axis_name="s")

    scratch = [
        pltpu.VMEM((2, bq), jnp.int32),
        pltpu.VMEM((2, bq, EMBED), jnp.float32),
        pltpu.VMEM((2, EMBED, bq), jnp.float32),
    ] + [pltpu.SemaphoreType.DMA] * 4

    @functools.partial(
        pl.kernel,
        out_type=jax.ShapeDtypeStruct((hist, EMBED, batch), jnp.float32),
        mesh=mesh,
        scratch_types=scratch,
        compiler_params=pltpu.CompilerParams(
            use_tc_tiling_on_sc=False, needs_layout_passes=False
        ),
    )
    def emb_kernel(idx_hbm, table_hbm, out_hbm, idxv, rows, outb, *sems):
        g_sems = sems[:2]
        o_sems = sems[2:]
        wid = lax.axis_index("s") * info.num_cores + lax.axis_index("c")
        t_base = wid * n_tasks

        def task_hb(k):
            t = t_base + k
            return t // n_bq, (t % n_bq) * bq

        def gather_start(k, s):
            h, b0 = task_hb(k)
            pltpu.sync_copy(idx_hbm.at[h, pl.ds(b0, bq)], idxv.at[s])
            return pltpu.async_copy(
                table_hbm.at[idxv.at[s]], rows.at[s], g_sems[s]
            )

        def out_start(k, s):
            h, b0 = task_hb(k)
            return pltpu.async_copy(
                outb.at[s], out_hbm.at[h, :, pl.ds(b0, bq)], o_sems[s]
            )

        def transpose(s):
            lanes = lax.iota(jnp.int32, EMBED)

            def body(j, _):
                row_idx = lanes + j * EMBED
                for e in range(EMBED):
                    col_idx = jnp.full((EMBED,), e, jnp.int32)
                    v = plsc.load_gather(rows.at[s], [row_idx, col_idx])
                    outb[s, e, pl.ds(j * EMBED, EMBED)] = v
                return 0

            lax.fori_loop(0, bq // EMBED, body, 0)

        g_cp = [None, None]
        o_cp = [None, None]
        for t in range(n_tasks):
            s = t % 2
            g_cp[s] = gather_start(t, s)
            if t > 0:
                sp = 1 - s
                g_cp[sp].wait()
                if o_cp[sp] is not None:
                    o_cp[sp].wait()
                transpose(sp)
                o_cp[sp] = out_start(t - 1, sp)
        s = (n_tasks - 1) % 2
        g_cp[s].wait()
        if o_cp[s] is not None:
            o_cp[s].wait()
        transpose(s)
        o_cp[s] = out_start(n_tasks - 1, s)
        for s in range(2):
            if o_cp[s] is not None:
                o_cp[s].wait()

    return emb_kernel


def kernel(x, table):
    batch, _, hist = x.shape
    idx_t = x[:, 0, :].T
    fn = _build(hist, batch, table.shape[0])
    out = fn(idx_t, table)
    return jnp.transpose(out, (2, 0, 1))
